# Initial kernel scaffold; baseline (speedup 1.0000x reference)
import jax
import jax.numpy as jnp
from jax.experimental import pallas as pl


def kernel(x, edge_index, W1, b1, W2, b2):
    raise NotImplementedError("write your pallas kernel here")


# speedup vs baseline: 15.1719x; 15.1719x over previous
import functools

import jax
import jax.numpy as jnp
from jax import lax
from jax.experimental import pallas as pl
from jax.experimental.pallas import tpu as pltpu
from jax.experimental.pallas import tpu_sc as plsc

N_NODES = 10000
N_EDGES = 320000
IN_DIM = 128
HID_DIM = 16
OUT_DIM = 2

NC = 2
NS = 16
NW = NC * NS
LANES = 16

N_PAD = 10112
RPT = N_PAD // NS
CHUNK = 128
NCH = 80
E_PAD = NW * NCH * CHUNK
KG = 8
NGRP = NCH // KG


def _mesh():
    return plsc.VectorSubcoreMesh(core_axis_name="c", subcore_axis_name="s")


_SC_PARAMS = pltpu.CompilerParams(use_tc_tiling_on_sc=False)


def _deg_body(dstw, degp, acc_sh, dst_v, ones_v, zbuf, sem_in, sem_sc):
    c = lax.axis_index("c")
    s = lax.axis_index("s")
    wid = c * NS + s

    in_cp = pltpu.async_copy(dstw.at[wid], dst_v, sem_in)

    def fill_ones(i, carry):
        ones_v[pl.ds(i * LANES, LANES)] = jnp.ones((LANES,), jnp.float32)
        return carry

    lax.fori_loop(0, CHUNK // LANES, fill_ones, 0)

    def fill_zero(i, carry):
        zbuf[pl.ds(i * LANES, LANES)] = jnp.zeros((LANES,), jnp.float32)
        return carry

    lax.fori_loop(0, RPT // LANES, fill_zero, 0)
    pltpu.sync_copy(zbuf, acc_sh.at[pl.ds(s * RPT, RPT)])
    in_cp.wait()
    plsc.subcore_barrier()

    def group(t, carry):
        cps = []
        for k in range(KG):
            j = t * KG + k
            cps.append(pltpu.async_copy(
                ones_v, acc_sh.at[dst_v.at[j]], sem_sc, add=True))
        for cp in cps:
            cp.wait()
        return carry

    lax.fori_loop(0, NGRP, group, 0)
    plsc.subcore_barrier()
    pltpu.sync_copy(acc_sh.at[pl.ds(s * RPT, RPT)],
                    degp.at[c, pl.ds(s * RPT, RPT)])


def _sc_degree(dstw):
    kern = pl.kernel(
        _deg_body,
        out_type=jax.ShapeDtypeStruct((NC, N_PAD), jnp.float32),
        mesh=_mesh(),
        scratch_types=[
            pltpu.VMEM_SHARED((N_PAD,), jnp.float32),
            pltpu.VMEM((NCH, CHUNK), jnp.int32),
            pltpu.VMEM((CHUNK,), jnp.float32),
            pltpu.VMEM((RPT,), jnp.float32),
            pltpu.SemaphoreType.DMA,
            pltpu.SemaphoreType.DMA,
        ],
        compiler_params=_SC_PARAMS,
    )
    return kern(dstw)


def _layer_body(d, table, srcw, dstw, outp, table_sh, acc_sh, src_v, dst_v,
                rows_v, sem_si, sem_di, sem_g, sem_s):
    c = lax.axis_index("c")
    s = lax.axis_index("s")
    wid = c * NS + s

    si_cp = pltpu.async_copy(srcw.at[wid], src_v, sem_si)
    di_cp = pltpu.async_copy(dstw.at[wid], dst_v, sem_di)
    pltpu.sync_copy(table.at[pl.ds(s * RPT, RPT)],
                    table_sh.at[pl.ds(s * RPT, RPT)])
    pltpu.sync_copy(table.at[pl.ds(s * RPT, RPT)],
                    acc_sh.at[pl.ds(s * RPT, RPT)])
    si_cp.wait()
    di_cp.wait()
    plsc.subcore_barrier()

    def group(t, carry):
        gcps = []
        for k in range(KG):
            j = t * KG + k
            gcps.append(pltpu.async_copy(
                table_sh.at[src_v.at[j]], rows_v.at[k], sem_g))
        for cp in gcps:
            cp.wait()
        scps = []
        for k in range(KG):
            j = t * KG + k
            scps.append(pltpu.async_copy(
                rows_v.at[k], acc_sh.at[dst_v.at[j]], sem_s, add=True))
        for cp in scps:
            cp.wait()
        return carry

    lax.fori_loop(0, NGRP, group, 0)
    plsc.subcore_barrier()
    pltpu.sync_copy(acc_sh.at[pl.ds(s * RPT, RPT)],
                    outp.at[c, pl.ds(s * RPT, RPT)])


def _sc_layer(table, srcw, dstw, d):
    kern = pl.kernel(
        functools.partial(_layer_body, d),
        out_type=jax.ShapeDtypeStruct((NC, N_PAD, d), jnp.float32),
        mesh=_mesh(),
        scratch_types=[
            pltpu.VMEM_SHARED((N_PAD, d), jnp.float32),
            pltpu.VMEM_SHARED((N_PAD, d), jnp.float32),
            pltpu.VMEM((NCH, CHUNK), jnp.int32),
            pltpu.VMEM((NCH, CHUNK), jnp.int32),
            pltpu.VMEM((KG, CHUNK, d), jnp.float32),
            pltpu.SemaphoreType.DMA,
            pltpu.SemaphoreType.DMA,
            pltpu.SemaphoreType.DMA,
            pltpu.SemaphoreType.DMA,
        ],
        compiler_params=_SC_PARAMS,
    )
    return kern(table, srcw, dstw)


NCH2 = 2 * NCH
NGRP2 = NCH2 // KG
E2_PAD = NW * NCH2 * CHUNK


def _flat_body(table, srcw, dstw, outp, table_sh, acc_sh, src_v, dst_v,
               rows_v, sem_si, sem_di, sem_g, sem_s):
    c = lax.axis_index("c")
    s = lax.axis_index("s")
    wid = c * NS + s
    rpt = (N_PAD * OUT_DIM) // NS

    si_cp = pltpu.async_copy(srcw.at[wid], src_v, sem_si)
    di_cp = pltpu.async_copy(dstw.at[wid], dst_v, sem_di)
    pltpu.sync_copy(table.at[pl.ds(s * rpt, rpt)],
                    table_sh.at[pl.ds(s * rpt, rpt)])
    pltpu.sync_copy(table.at[pl.ds(s * rpt, rpt)],
                    acc_sh.at[pl.ds(s * rpt, rpt)])
    si_cp.wait()
    di_cp.wait()
    plsc.subcore_barrier()

    def group(t, carry):
        gcps = []
        for k in range(KG):
            j = t * KG + k
            gcps.append(pltpu.async_copy(
                table_sh.at[src_v.at[j]], rows_v.at[k], sem_g))
        for cp in gcps:
            cp.wait()
        scps = []
        for k in range(KG):
            j = t * KG + k
            scps.append(pltpu.async_copy(
                rows_v.at[k], acc_sh.at[dst_v.at[j]], sem_s, add=True))
        for cp in scps:
            cp.wait()
        return carry

    lax.fori_loop(0, NGRP2, group, 0)
    plsc.subcore_barrier()
    pltpu.sync_copy(acc_sh.at[pl.ds(s * rpt, rpt)],
                    outp.at[c, pl.ds(s * rpt, rpt)])


def _sc_layer_flat(table_flat, srcw2, dstw2):
    m = N_PAD * OUT_DIM
    kern = pl.kernel(
        _flat_body,
        out_type=jax.ShapeDtypeStruct((NC, m), jnp.float32),
        mesh=_mesh(),
        scratch_types=[
            pltpu.VMEM_SHARED((m,), jnp.float32),
            pltpu.VMEM_SHARED((m,), jnp.float32),
            pltpu.VMEM((NCH2, CHUNK), jnp.int32),
            pltpu.VMEM((NCH2, CHUNK), jnp.int32),
            pltpu.VMEM((KG, CHUNK), jnp.float32),
            pltpu.SemaphoreType.DMA,
            pltpu.SemaphoreType.DMA,
            pltpu.SemaphoreType.DMA,
            pltpu.SemaphoreType.DMA,
        ],
        compiler_params=_SC_PARAMS,
    )
    return kern(table_flat, srcw2, dstw2)


def _tc1_body(x_ref, w1_ref, degp_ref, g1_ref, dinv_ref):
    deg = degp_ref[0] + degp_ref[1] + 1.0
    dinv = lax.rsqrt(deg)
    h = jnp.dot(x_ref[...], w1_ref[...], preferred_element_type=jnp.float32)
    g1_ref[...] = h * dinv[:, None]
    dinv_ref[...] = dinv[:, None]


def _tc1(x_pad, W1, degp):
    return pl.pallas_call(
        _tc1_body,
        out_shape=[
            jax.ShapeDtypeStruct((N_PAD, HID_DIM), jnp.float32),
            jax.ShapeDtypeStruct((N_PAD, 1), jnp.float32),
        ],
    )(x_pad, W1, degp)


def _tc2_body(s1p_ref, g1_ref, dinv_ref, b1_ref, w2_ref, g2_ref):
    dinv = dinv_ref[...]
    ssum = s1p_ref[0] + s1p_ref[1] - g1_ref[...]
    h1o = jnp.maximum(ssum * dinv + b1_ref[...], 0.0)
    h2 = jnp.dot(h1o, w2_ref[...], preferred_element_type=jnp.float32)
    g2_ref[...] = h2 * dinv


def _tc2(s1p, g1, dinv, b1, W2):
    return pl.pallas_call(
        _tc2_body,
        out_shape=jax.ShapeDtypeStruct((N_PAD, OUT_DIM), jnp.float32),
    )(s1p, g1, dinv, b1.reshape(1, HID_DIM), W2)


def _tc3_body(s2p_ref, g2_ref, dinv_ref, b2_ref, out_ref):
    ssum = s2p_ref[0] + s2p_ref[1] - g2_ref[...]
    out_ref[...] = ssum * dinv_ref[...] + b2_ref[...]


def _tc3(s2p, g2, dinv, b2):
    return pl.pallas_call(
        _tc3_body,
        out_shape=jax.ShapeDtypeStruct((N_PAD, OUT_DIM), jnp.float32),
    )(s2p, g2, dinv, b2.reshape(1, OUT_DIM))


def kernel(x, edge_index, W1, b1, W2, b2):
    src = edge_index[0].astype(jnp.int32)
    dst = edge_index[1].astype(jnp.int32)
    pad = jnp.full((E_PAD - N_EDGES,), N_NODES, dtype=jnp.int32)
    srcw = jnp.concatenate([src, pad]).reshape(NW, NCH, CHUNK)
    dstw = jnp.concatenate([dst, pad]).reshape(NW, NCH, CHUNK)
    x_pad = jnp.zeros((N_PAD, IN_DIM), jnp.float32).at[:N_NODES].set(x)

    two = jnp.arange(OUT_DIM, dtype=jnp.int32)
    pad2 = jnp.full((E2_PAD - OUT_DIM * N_EDGES,), OUT_DIM * N_NODES,
                    dtype=jnp.int32)
    src2 = (src[:, None] * OUT_DIM + two).reshape(-1)
    dst2 = (dst[:, None] * OUT_DIM + two).reshape(-1)
    srcw2 = jnp.concatenate([src2, pad2]).reshape(NW, NCH2, CHUNK)
    dstw2 = jnp.concatenate([dst2, pad2]).reshape(NW, NCH2, CHUNK)

    degp = _sc_degree(dstw)
    g1, dinv = _tc1(x_pad, W1, degp)
    s1p = _sc_layer(g1, srcw, dstw, HID_DIM)
    g2 = _tc2(s1p, g1, dinv, b1, W2)
    s2p = _sc_layer_flat(g2.reshape(-1), srcw2, dstw2).reshape(
        NC, N_PAD, OUT_DIM)
    out = _tc3(s2p, g2, dinv, b2)
    return out[:N_NODES]


# speedup vs baseline: 72.0796x; 4.7508x over previous
import functools

import jax
import jax.numpy as jnp
from jax import lax
from jax.experimental import pallas as pl
from jax.experimental.pallas import tpu as pltpu
from jax.experimental.pallas import tpu_sc as plsc

N_NODES = 10000
N_EDGES = 320000
IN_DIM = 128
HID_DIM = 16
OUT_DIM = 2

NC = 2
NS = 16
NW = NC * NS
LANES = 16

N_PAD = 10112
RPT = N_PAD // NS
CHUNK = 128
ROWS = N_EDGES // CHUNK
NCH = ROWS // NW
XTRA = ROWS - NCH * NW
KG = 6
NGRP = NCH // KG


def _mesh():
    return plsc.VectorSubcoreMesh(core_axis_name="c", subcore_axis_name="s")


_SC_PARAMS = pltpu.CompilerParams(use_tc_tiling_on_sc=False)


def _deg_body(eidx, degp, acc_sh, dst_v, dstt_v, ones_v, zbuf, sem_in,
              sem_sc):
    c = lax.axis_index("c")
    s = lax.axis_index("s")
    wid = c * NS + s

    in_cp = pltpu.async_copy(eidx.at[1, pl.ds(wid * NCH, NCH)], dst_v, sem_in)
    trow = NW * NCH + jnp.minimum(wid, XTRA - 1)
    t_cp = pltpu.async_copy(eidx.at[1, pl.ds(trow, 1)], dstt_v, sem_in)

    def fill_ones(i, carry):
        ones_v[pl.ds(i * LANES, LANES)] = jnp.ones((LANES,), jnp.float32)
        return carry

    lax.fori_loop(0, CHUNK // LANES, fill_ones, 0)

    def fill_zero(i, carry):
        zbuf[pl.ds(i * LANES, LANES)] = jnp.zeros((LANES,), jnp.float32)
        return carry

    lax.fori_loop(0, RPT // LANES, fill_zero, 0)
    pltpu.sync_copy(zbuf, acc_sh.at[pl.ds(s * RPT, RPT)])
    in_cp.wait()
    t_cp.wait()
    plsc.subcore_barrier()

    def group(t, carry):
        cps = []
        for k in range(KG):
            j = t * KG + k
            cps.append(pltpu.async_copy(
                ones_v, acc_sh.at[dst_v.at[j]], sem_sc, add=True))
        for cp in cps:
            cp.wait()
        return carry

    lax.fori_loop(0, NGRP, group, 0)

    @pl.when(wid < XTRA)
    def _tail():
        pltpu.async_copy(ones_v, acc_sh.at[dstt_v.at[0]], sem_sc,
                         add=True).wait()

    plsc.subcore_barrier()
    pltpu.sync_copy(acc_sh.at[pl.ds(s * RPT, RPT)],
                    degp.at[c, pl.ds(s * RPT, RPT)])


def _sc_degree(eidx):
    kern = pl.kernel(
        _deg_body,
        out_type=jax.ShapeDtypeStruct((NC, N_PAD), jnp.float32),
        mesh=_mesh(),
        scratch_types=[
            pltpu.VMEM_SHARED((N_PAD,), jnp.float32),
            pltpu.VMEM((NCH, CHUNK), jnp.int32),
            pltpu.VMEM((1, CHUNK), jnp.int32),
            pltpu.VMEM((CHUNK,), jnp.float32),
            pltpu.VMEM((RPT,), jnp.float32),
            pltpu.SemaphoreType.DMA,
            pltpu.SemaphoreType.DMA,
        ],
        compiler_params=_SC_PARAMS,
    )
    return kern(eidx)


def _layer_body(eidx, table, outp, table_sh, acc_sh, src_v, dst_v, srct_v,
                dstt_v, rows_v, rowt_v, sem_i, sem_g, sem_s):
    c = lax.axis_index("c")
    s = lax.axis_index("s")
    wid = c * NS + s

    si_cp = pltpu.async_copy(eidx.at[0, pl.ds(wid * NCH, NCH)], src_v, sem_i)
    di_cp = pltpu.async_copy(eidx.at[1, pl.ds(wid * NCH, NCH)], dst_v, sem_i)
    trow = NW * NCH + jnp.minimum(wid, XTRA - 1)
    st_cp = pltpu.async_copy(eidx.at[0, pl.ds(trow, 1)], srct_v, sem_i)
    dt_cp = pltpu.async_copy(eidx.at[1, pl.ds(trow, 1)], dstt_v, sem_i)
    pltpu.sync_copy(table.at[pl.ds(s * RPT, RPT)],
                    table_sh.at[pl.ds(s * RPT, RPT)])
    pltpu.sync_copy(table.at[pl.ds(s * RPT, RPT)],
                    acc_sh.at[pl.ds(s * RPT, RPT)])
    si_cp.wait()
    di_cp.wait()
    st_cp.wait()
    dt_cp.wait()
    plsc.subcore_barrier()

    def group(t, carry):
        gcps = []
        for k in range(KG):
            j = t * KG + k
            gcps.append(pltpu.async_copy(
                table_sh.at[src_v.at[j]], rows_v.at[k], sem_g))
        for cp in gcps:
            cp.wait()
        scps = []
        for k in range(KG):
            j = t * KG + k
            scps.append(pltpu.async_copy(
                rows_v.at[k], acc_sh.at[dst_v.at[j]], sem_s, add=True))
        for cp in scps:
            cp.wait()
        return carry

    lax.fori_loop(0, NGRP, group, 0)

    @pl.when(wid < XTRA)
    def _tail():
        pltpu.async_copy(table_sh.at[srct_v.at[0]], rowt_v, sem_g).wait()
        pltpu.async_copy(rowt_v, acc_sh.at[dstt_v.at[0]], sem_s,
                         add=True).wait()

    plsc.subcore_barrier()
    pltpu.sync_copy(acc_sh.at[pl.ds(s * RPT, RPT)],
                    outp.at[c, pl.ds(s * RPT, RPT)])


def _sc_layer1(eidx, table):
    kern = pl.kernel(
        _layer_body,
        out_type=jax.ShapeDtypeStruct((NC, N_PAD, HID_DIM), jnp.float32),
        mesh=_mesh(),
        scratch_types=[
            pltpu.VMEM_SHARED((N_PAD, HID_DIM), jnp.float32),
            pltpu.VMEM_SHARED((N_PAD, HID_DIM), jnp.float32),
            pltpu.VMEM((NCH, CHUNK), jnp.int32),
            pltpu.VMEM((NCH, CHUNK), jnp.int32),
            pltpu.VMEM((1, CHUNK), jnp.int32),
            pltpu.VMEM((1, CHUNK), jnp.int32),
            pltpu.VMEM((KG, CHUNK, HID_DIM), jnp.float32),
            pltpu.VMEM((CHUNK, HID_DIM), jnp.float32),
            pltpu.SemaphoreType.DMA,
            pltpu.SemaphoreType.DMA,
            pltpu.SemaphoreType.DMA,
        ],
        compiler_params=_SC_PARAMS,
    )
    return kern(eidx, table)


def _flat_body(eidx, table, outp, table_sh, acc_sh, src_v, dst_v, srct_v,
               dstt_v, rows_v, rowt_v, sem_i, sem_g, sem_s):
    c = lax.axis_index("c")
    s = lax.axis_index("s")
    wid = c * NS + s
    rpt = (N_PAD * OUT_DIM) // NS

    si_cp = pltpu.async_copy(eidx.at[0, pl.ds(wid * NCH, NCH)], src_v, sem_i)
    di_cp = pltpu.async_copy(eidx.at[1, pl.ds(wid * NCH, NCH)], dst_v, sem_i)
    trow = NW * NCH + jnp.minimum(wid, XTRA - 1)
    st_cp = pltpu.async_copy(eidx.at[0, pl.ds(trow, 1)], srct_v, sem_i)
    dt_cp = pltpu.async_copy(eidx.at[1, pl.ds(trow, 1)], dstt_v, sem_i)
    pltpu.sync_copy(table.at[pl.ds(s * rpt, rpt)],
                    table_sh.at[pl.ds(s * rpt, rpt)])
    pltpu.sync_copy(table.at[pl.ds(s * rpt, rpt)],
                    acc_sh.at[pl.ds(s * rpt, rpt)])
    si_cp.wait()
    di_cp.wait()
    st_cp.wait()
    dt_cp.wait()
    plsc.subcore_barrier()

    def one_chunk(src_idx, dst_idx, rbuf_e, rbuf_o, gcps, scps):
        gcps.append(pltpu.async_copy(
            table_sh.at[pl.ds(0, N_PAD)].at[src_idx], rbuf_e, sem_g))
        gcps.append(pltpu.async_copy(
            table_sh.at[pl.ds(N_PAD, N_PAD)].at[src_idx], rbuf_o, sem_g))
        scps.append((rbuf_e, 0, dst_idx))
        scps.append((rbuf_o, N_PAD, dst_idx))

    def group(t, carry):
        gcps, scps = [], []
        for k in range(KG):
            j = t * KG + k
            one_chunk(src_v.at[j], dst_v.at[j], rows_v.at[2 * k],
                      rows_v.at[2 * k + 1], gcps, scps)
        for cp in gcps:
            cp.wait()
        out = []
        for rbuf, off, dst_idx in scps:
            out.append(pltpu.async_copy(
                rbuf, acc_sh.at[pl.ds(off, N_PAD)].at[dst_idx], sem_s,
                add=True))
        for cp in out:
            cp.wait()
        return carry

    lax.fori_loop(0, NGRP, group, 0)

    @pl.when(wid < XTRA)
    def _tail():
        for off, rbuf in ((0, rowt_v.at[0]), (N_PAD, rowt_v.at[1])):
            pltpu.async_copy(
                table_sh.at[pl.ds(off, N_PAD)].at[srct_v.at[0]], rbuf,
                sem_g).wait()
            pltpu.async_copy(
                rbuf, acc_sh.at[pl.ds(off, N_PAD)].at[dstt_v.at[0]], sem_s,
                add=True).wait()

    plsc.subcore_barrier()
    pltpu.sync_copy(acc_sh.at[pl.ds(s * rpt, rpt)],
                    outp.at[c, pl.ds(s * rpt, rpt)])


def _sc_layer2(eidx, table_flat):
    m = N_PAD * OUT_DIM
    kern = pl.kernel(
        _flat_body,
        out_type=jax.ShapeDtypeStruct((NC, m), jnp.float32),
        mesh=_mesh(),
        scratch_types=[
            pltpu.VMEM_SHARED((m,), jnp.float32),
            pltpu.VMEM_SHARED((m,), jnp.float32),
            pltpu.VMEM((NCH, CHUNK), jnp.int32),
            pltpu.VMEM((NCH, CHUNK), jnp.int32),
            pltpu.VMEM((1, CHUNK), jnp.int32),
            pltpu.VMEM((1, CHUNK), jnp.int32),
            pltpu.VMEM((2 * KG, CHUNK), jnp.float32),
            pltpu.VMEM((2, CHUNK), jnp.float32),
            pltpu.SemaphoreType.DMA,
            pltpu.SemaphoreType.DMA,
            pltpu.SemaphoreType.DMA,
        ],
        compiler_params=_SC_PARAMS,
    )
    return kern(eidx, table_flat)


def _tc1_body(x_ref, w1_ref, degp_ref, g1_ref, dinv_ref):
    deg = degp_ref[0] + degp_ref[1] + 1.0
    dinv = lax.rsqrt(deg)
    h = jnp.dot(x_ref[...], w1_ref[...], preferred_element_type=jnp.float32)
    g1_ref[...] = h * dinv[:, None]
    dinv_ref[...] = dinv[:, None]


def _tc1(x_pad, W1, degp):
    return pl.pallas_call(
        _tc1_body,
        out_shape=[
            jax.ShapeDtypeStruct((N_PAD, HID_DIM), jnp.float32),
            jax.ShapeDtypeStruct((N_PAD, 1), jnp.float32),
        ],
    )(x_pad, W1, degp)


def _tc2_body(s1p_ref, g1_ref, dinv_ref, b1_ref, w2_ref, g2_ref):
    dinv = dinv_ref[...]
    ssum = s1p_ref[0] + s1p_ref[1] - g1_ref[...]
    h1o = jnp.maximum(ssum * dinv + b1_ref[...], 0.0)
    h2 = jnp.dot(h1o, w2_ref[...], preferred_element_type=jnp.float32)
    g2 = h2 * dinv
    g2_ref[pl.ds(0, N_PAD)] = g2[:, 0]
    g2_ref[pl.ds(N_PAD, N_PAD)] = g2[:, 1]


def _tc2(s1p, g1, dinv, b1, W2):
    return pl.pallas_call(
        _tc2_body,
        out_shape=jax.ShapeDtypeStruct((N_PAD * OUT_DIM,), jnp.float32),
    )(s1p, g1, dinv, b1.reshape(1, HID_DIM), W2)


def _tc3_body(s2p_ref, g2_ref, dinv_ref, b2_ref, out_ref):
    se = (s2p_ref[0, pl.ds(0, N_PAD)] + s2p_ref[1, pl.ds(0, N_PAD)]
          - g2_ref[pl.ds(0, N_PAD)])
    so = (s2p_ref[0, pl.ds(N_PAD, N_PAD)] + s2p_ref[1, pl.ds(N_PAD, N_PAD)]
          - g2_ref[pl.ds(N_PAD, N_PAD)])
    dinv = dinv_ref[...][:, 0]
    out_ref[...] = (
        jnp.stack([se, so], axis=1) * dinv[:, None] + b2_ref[...])


def _tc3(s2p, g2_flat, dinv, b2):
    return pl.pallas_call(
        _tc3_body,
        out_shape=jax.ShapeDtypeStruct((N_PAD, OUT_DIM), jnp.float32),
    )(s2p, g2_flat, dinv, b2.reshape(1, OUT_DIM))


def kernel(x, edge_index, W1, b1, W2, b2):
    eidx = edge_index.astype(jnp.int32).reshape(2, ROWS, CHUNK)
    x_pad = jnp.zeros((N_PAD, IN_DIM), jnp.float32).at[:N_NODES].set(x)

    degp = _sc_degree(eidx)
    g1, dinv = _tc1(x_pad, W1, degp)
    s1p = _sc_layer1(eidx, g1)
    g2 = _tc2(s1p, g1, dinv, b1, W2)
    s2p = _sc_layer2(eidx, g2)
    out = _tc3(s2p, g2, dinv, b2)
    return out[:N_NODES]


# speedup vs baseline: 87.8765x; 1.2192x over previous
import jax
import jax.numpy as jnp
from jax import lax
from jax.experimental import pallas as pl
from jax.experimental.pallas import tpu as pltpu
from jax.experimental.pallas import tpu_sc as plsc

N_NODES = 10000
N_EDGES = 320000
IN_DIM = 128
HID_DIM = 16
OUT_DIM = 2

NC = 2
NS = 16
NW = NC * NS
LANES = 16

N_PAD = 10112
RPT = N_PAD // NS
CHUNK = 128
ROWS = N_EDGES // CHUNK
NCH = ROWS // NW
XTRA = ROWS - NCH * NW
KG = 13
NGRP = NCH // KG


def _mesh():
    return plsc.VectorSubcoreMesh(core_axis_name="c", subcore_axis_name="s")


_SC_PARAMS = pltpu.CompilerParams(use_tc_tiling_on_sc=False)


def _deg_body(eidx, degp, acc_sh, dst_v, dstt_v, ones_v, zbuf, sem_in,
              sem_sc):
    c = lax.axis_index("c")
    s = lax.axis_index("s")
    wid = c * NS + s

    in_cp = pltpu.async_copy(eidx.at[1, pl.ds(wid * NCH, NCH)], dst_v, sem_in)
    trow = NW * NCH + jnp.minimum(wid, XTRA - 1)
    t_cp = pltpu.async_copy(eidx.at[1, pl.ds(trow, 1)], dstt_v, sem_in)

    def fill_ones(i, carry):
        ones_v[pl.ds(i * LANES, LANES)] = jnp.ones((LANES,), jnp.float32)
        return carry

    lax.fori_loop(0, CHUNK // LANES, fill_ones, 0)

    def fill_zero(i, carry):
        zbuf[pl.ds(i * LANES, LANES)] = jnp.zeros((LANES,), jnp.float32)
        return carry

    lax.fori_loop(0, RPT // LANES, fill_zero, 0)
    pltpu.sync_copy(zbuf, acc_sh.at[pl.ds(s * RPT, RPT)])
    in_cp.wait()
    t_cp.wait()
    plsc.subcore_barrier()

    def fire(j, carry):
        pltpu.async_copy(ones_v, acc_sh.at[dst_v.at[j]], sem_sc, add=True)
        return carry

    lax.fori_loop(0, NCH, fire, 0)

    @pl.when(wid < XTRA)
    def _tail():
        pltpu.async_copy(ones_v, acc_sh.at[dstt_v.at[0]], sem_sc, add=True)

    def drain(j, carry):
        pltpu.make_async_copy(ones_v, acc_sh.at[dst_v.at[0]], sem_sc).wait()
        return carry

    lax.fori_loop(0, NCH, drain, 0)

    @pl.when(wid < XTRA)
    def _tail_drain():
        pltpu.make_async_copy(ones_v, acc_sh.at[dst_v.at[0]], sem_sc).wait()

    plsc.subcore_barrier()
    pltpu.sync_copy(acc_sh.at[pl.ds(s * RPT, RPT)],
                    degp.at[c, pl.ds(s * RPT, RPT)])


def _sc_degree(eidx):
    kern = pl.kernel(
        _deg_body,
        out_type=jax.ShapeDtypeStruct((NC, N_PAD), jnp.float32),
        mesh=_mesh(),
        scratch_types=[
            pltpu.VMEM_SHARED((N_PAD,), jnp.float32),
            pltpu.VMEM((NCH, CHUNK), jnp.int32),
            pltpu.VMEM((1, CHUNK), jnp.int32),
            pltpu.VMEM((CHUNK,), jnp.float32),
            pltpu.VMEM((RPT,), jnp.float32),
            pltpu.SemaphoreType.DMA,
            pltpu.SemaphoreType.DMA,
        ],
        compiler_params=_SC_PARAMS,
    )
    return kern(eidx)


def _layer_body(eidx, table, outp, table_sh, acc_sh, src_v, dst_v, srct_v,
                dstt_v, rows_v, rowt_v, sem_i, sem_g, sem_s):
    c = lax.axis_index("c")
    s = lax.axis_index("s")
    wid = c * NS + s

    si_cp = pltpu.async_copy(eidx.at[0, pl.ds(wid * NCH, NCH)], src_v, sem_i)
    di_cp = pltpu.async_copy(eidx.at[1, pl.ds(wid * NCH, NCH)], dst_v, sem_i)
    trow = NW * NCH + jnp.minimum(wid, XTRA - 1)
    st_cp = pltpu.async_copy(eidx.at[0, pl.ds(trow, 1)], srct_v, sem_i)
    dt_cp = pltpu.async_copy(eidx.at[1, pl.ds(trow, 1)], dstt_v, sem_i)
    pltpu.sync_copy(table.at[pl.ds(s * RPT, RPT)],
                    table_sh.at[pl.ds(s * RPT, RPT)])
    pltpu.sync_copy(table.at[pl.ds(s * RPT, RPT)],
                    acc_sh.at[pl.ds(s * RPT, RPT)])
    si_cp.wait()
    di_cp.wait()
    st_cp.wait()
    dt_cp.wait()
    plsc.subcore_barrier()

    def fire_g(g, half):
        for k in range(KG):
            pltpu.async_copy(table_sh.at[src_v.at[g * KG + k]],
                             rows_v.at[half * KG + k], sem_g)

    def drain_g():
        for _ in range(KG):
            pltpu.make_async_copy(table_sh.at[src_v.at[0]], rows_v.at[0],
                                  sem_g).wait()

    def fire_s(g, half):
        for k in range(KG):
            pltpu.async_copy(rows_v.at[half * KG + k],
                             acc_sh.at[dst_v.at[g * KG + k]], sem_s,
                             add=True)

    def drain_s():
        for _ in range(KG):
            pltpu.make_async_copy(rows_v.at[0], acc_sh.at[dst_v.at[0]],
                                  sem_s).wait()

    fire_g(0, 0)

    def body(t, carry):
        g = 2 * t
        drain_g()
        fire_g(g + 1, 1)
        fire_s(g, 0)
        drain_s()
        drain_g()

        @pl.when(g + 2 < NGRP)
        def _():
            fire_g(g + 2, 0)

        fire_s(g + 1, 1)
        drain_s()
        return carry

    lax.fori_loop(0, NGRP // 2, body, 0)

    @pl.when(wid < XTRA)
    def _tail():
        pltpu.async_copy(table_sh.at[srct_v.at[0]], rowt_v, sem_g).wait()
        pltpu.async_copy(rowt_v, acc_sh.at[dstt_v.at[0]], sem_s,
                         add=True).wait()

    plsc.subcore_barrier()
    pltpu.sync_copy(acc_sh.at[pl.ds(s * RPT, RPT)],
                    outp.at[c, pl.ds(s * RPT, RPT)])


def _sc_layer1(eidx, table):
    kern = pl.kernel(
        _layer_body,
        out_type=jax.ShapeDtypeStruct((NC, N_PAD, HID_DIM), jnp.float32),
        mesh=_mesh(),
        scratch_types=[
            pltpu.VMEM_SHARED((N_PAD, HID_DIM), jnp.float32),
            pltpu.VMEM_SHARED((N_PAD, HID_DIM), jnp.float32),
            pltpu.VMEM((NCH, CHUNK), jnp.int32),
            pltpu.VMEM((NCH, CHUNK), jnp.int32),
            pltpu.VMEM((1, CHUNK), jnp.int32),
            pltpu.VMEM((1, CHUNK), jnp.int32),
            pltpu.VMEM((2 * KG, CHUNK, HID_DIM), jnp.float32),
            pltpu.VMEM((CHUNK, HID_DIM), jnp.float32),
            pltpu.SemaphoreType.DMA,
            pltpu.SemaphoreType.DMA,
            pltpu.SemaphoreType.DMA,
        ],
        compiler_params=_SC_PARAMS,
    )
    return kern(eidx, table)


def _flat_body(eidx, table, outp, table_sh, acc_sh, src_v, dst_v, srct_v,
               dstt_v, rows_v, rowt_v, sem_i, sem_g, sem_s):
    c = lax.axis_index("c")
    s = lax.axis_index("s")
    wid = c * NS + s
    rpt = (N_PAD * OUT_DIM) // NS

    si_cp = pltpu.async_copy(eidx.at[0, pl.ds(wid * NCH, NCH)], src_v, sem_i)
    di_cp = pltpu.async_copy(eidx.at[1, pl.ds(wid * NCH, NCH)], dst_v, sem_i)
    trow = NW * NCH + jnp.minimum(wid, XTRA - 1)
    st_cp = pltpu.async_copy(eidx.at[0, pl.ds(trow, 1)], srct_v, sem_i)
    dt_cp = pltpu.async_copy(eidx.at[1, pl.ds(trow, 1)], dstt_v, sem_i)
    pltpu.sync_copy(table.at[pl.ds(s * rpt, rpt)],
                    table_sh.at[pl.ds(s * rpt, rpt)])
    pltpu.sync_copy(table.at[pl.ds(s * rpt, rpt)],
                    acc_sh.at[pl.ds(s * rpt, rpt)])
    si_cp.wait()
    di_cp.wait()
    st_cp.wait()
    dt_cp.wait()
    plsc.subcore_barrier()

    def fire_g(g, half):
        for k in range(KG):
            j = g * KG + k
            for h in range(OUT_DIM):
                pltpu.async_copy(
                    table_sh.at[pl.ds(h * N_PAD, N_PAD)].at[src_v.at[j]],
                    rows_v.at[half * KG + k, h], sem_g)

    def drain_g():
        for _ in range(OUT_DIM * KG):
            pltpu.make_async_copy(
                table_sh.at[pl.ds(0, N_PAD)].at[src_v.at[0]],
                rows_v.at[0, 0], sem_g).wait()

    def fire_s(g, half):
        for k in range(KG):
            j = g * KG + k
            for h in range(OUT_DIM):
                pltpu.async_copy(
                    rows_v.at[half * KG + k, h],
                    acc_sh.at[pl.ds(h * N_PAD, N_PAD)].at[dst_v.at[j]],
                    sem_s, add=True)

    def drain_s():
        for _ in range(OUT_DIM * KG):
            pltpu.make_async_copy(
                rows_v.at[0, 0],
                acc_sh.at[pl.ds(0, N_PAD)].at[dst_v.at[0]], sem_s).wait()

    fire_g(0, 0)

    def body(t, carry):
        g = 2 * t
        drain_g()
        fire_g(g + 1, 1)
        fire_s(g, 0)
        drain_s()
        drain_g()

        @pl.when(g + 2 < NGRP)
        def _():
            fire_g(g + 2, 0)

        fire_s(g + 1, 1)
        drain_s()
        return carry

    lax.fori_loop(0, NGRP // 2, body, 0)

    @pl.when(wid < XTRA)
    def _tail():
        for h in range(OUT_DIM):
            pltpu.async_copy(
                table_sh.at[pl.ds(h * N_PAD, N_PAD)].at[srct_v.at[0]],
                rowt_v.at[h], sem_g).wait()
            pltpu.async_copy(
                rowt_v.at[h],
                acc_sh.at[pl.ds(h * N_PAD, N_PAD)].at[dstt_v.at[0]], sem_s,
                add=True).wait()

    plsc.subcore_barrier()
    pltpu.sync_copy(acc_sh.at[pl.ds(s * rpt, rpt)],
                    outp.at[c, pl.ds(s * rpt, rpt)])


def _sc_layer2(eidx, table_flat):
    m = N_PAD * OUT_DIM
    kern = pl.kernel(
        _flat_body,
        out_type=jax.ShapeDtypeStruct((NC, m), jnp.float32),
        mesh=_mesh(),
        scratch_types=[
            pltpu.VMEM_SHARED((m,), jnp.float32),
            pltpu.VMEM_SHARED((m,), jnp.float32),
            pltpu.VMEM((NCH, CHUNK), jnp.int32),
            pltpu.VMEM((NCH, CHUNK), jnp.int32),
            pltpu.VMEM((1, CHUNK), jnp.int32),
            pltpu.VMEM((1, CHUNK), jnp.int32),
            pltpu.VMEM((2 * KG, OUT_DIM, CHUNK), jnp.float32),
            pltpu.VMEM((OUT_DIM, CHUNK), jnp.float32),
            pltpu.SemaphoreType.DMA,
            pltpu.SemaphoreType.DMA,
            pltpu.SemaphoreType.DMA,
        ],
        compiler_params=_SC_PARAMS,
    )
    return kern(eidx, table_flat)


def _tc1a_body(x_ref, w1_ref, h_ref):
    h_ref[pl.ds(0, N_NODES), :] = jnp.dot(
        x_ref[...], w1_ref[...], preferred_element_type=jnp.float32)


def _tc1a(x, W1):
    return pl.pallas_call(
        _tc1a_body,
        out_shape=jax.ShapeDtypeStruct((N_PAD, HID_DIM), jnp.float32),
    )(x, W1)


def _tc1b_body(h_ref, degp_ref, g1_ref, dinv_ref, dinvt_ref):
    deg = degp_ref[0] + degp_ref[1] + 1.0
    dt = lax.rsqrt(deg)
    dinvt_ref[...] = dt[None, :]
    dsub = dt[:, None]
    dinv_ref[...] = dsub
    g1_ref[...] = h_ref[...] * dsub


def _tc1b(h, degp):
    return pl.pallas_call(
        _tc1b_body,
        out_shape=[
            jax.ShapeDtypeStruct((N_PAD, HID_DIM), jnp.float32),
            jax.ShapeDtypeStruct((N_PAD, 1), jnp.float32),
            jax.ShapeDtypeStruct((1, N_PAD), jnp.float32),
        ],
    )(h, degp)


def _tc2_body(s1p_ref, g1_ref, dinv_ref, dinvt_ref, b1_ref, w2t_ref,
              g2_ref):
    ssum = s1p_ref[0] + s1p_ref[1] - g1_ref[...]
    h1o = jnp.maximum(ssum * dinv_ref[...] + b1_ref[...], 0.0)
    g2t = lax.dot_general(
        w2t_ref[...], h1o, (((1,), (1,)), ((), ())),
        preferred_element_type=jnp.float32) * dinvt_ref[...]
    g2_ref[pl.ds(0, N_PAD)] = g2t[0]
    g2_ref[pl.ds(N_PAD, N_PAD)] = g2t[1]


def _tc2(s1p, g1, dinv, dinvt, b1, W2t):
    return pl.pallas_call(
        _tc2_body,
        out_shape=jax.ShapeDtypeStruct((N_PAD * OUT_DIM,), jnp.float32),
    )(s1p, g1, dinv, dinvt, b1.reshape(1, HID_DIM), W2t)


def _tc3_body(s2p_ref, g2_ref, dinvt_ref, b2_ref, out_ref):
    se = (s2p_ref[0, pl.ds(0, N_PAD)] + s2p_ref[1, pl.ds(0, N_PAD)]
          - g2_ref[pl.ds(0, N_PAD)])
    so = (s2p_ref[0, pl.ds(N_PAD, N_PAD)] + s2p_ref[1, pl.ds(N_PAD, N_PAD)]
          - g2_ref[pl.ds(N_PAD, N_PAD)])
    outt = jnp.stack([se, so], axis=0) * dinvt_ref[...] + b2_ref[...]
    out_ref[...] = outt.T


def _tc3(s2p, g2_flat, dinvt, b2):
    return pl.pallas_call(
        _tc3_body,
        out_shape=jax.ShapeDtypeStruct((N_PAD, OUT_DIM), jnp.float32),
    )(s2p, g2_flat, dinvt, b2.reshape(OUT_DIM, 1))


def kernel(x, edge_index, W1, b1, W2, b2):
    eidx = edge_index.astype(jnp.int32).reshape(2, ROWS, CHUNK)

    degp = _sc_degree(eidx)
    h1 = _tc1a(x, W1)
    g1, dinv, dinvt = _tc1b(h1, degp)
    s1p = _sc_layer1(eidx, g1)
    g2 = _tc2(s1p, g1, dinv, dinvt, b1, W2.T)
    s2p = _sc_layer2(eidx, g2)
    out = _tc3(s2p, g2, dinvt, b2)
    return out[:N_NODES]


# speedup vs baseline: 89.1036x; 1.0140x over previous
import jax
import jax.numpy as jnp
from jax import lax
from jax.experimental import pallas as pl
from jax.experimental.pallas import tpu as pltpu
from jax.experimental.pallas import tpu_sc as plsc

N_NODES = 10000
N_EDGES = 320000
IN_DIM = 128
HID_DIM = 16
OUT_DIM = 2

NC = 2
NS = 16
NW = NC * NS
LANES = 16

N_PAD = 10112
RPT = N_PAD // NS
CHUNK = 128
ROWS = N_EDGES // CHUNK
NCH = ROWS // NW
XTRA = ROWS - NCH * NW
KG = 13
NGRP = NCH // KG


def _mesh():
    return plsc.VectorSubcoreMesh(core_axis_name="c", subcore_axis_name="s")


_SC_PARAMS = pltpu.CompilerParams(use_tc_tiling_on_sc=False)


def _deg_body(eidx, degp, acc_sh, dst_v, dstt_v, ones_v, zbuf, sem_in,
              sem_sc):
    c = lax.axis_index("c")
    s = lax.axis_index("s")
    wid = c * NS + s

    in_cp = pltpu.async_copy(eidx.at[1, pl.ds(wid * NCH, NCH)], dst_v, sem_in)
    trow = NW * NCH + jnp.minimum(wid, XTRA - 1)
    t_cp = pltpu.async_copy(eidx.at[1, pl.ds(trow, 1)], dstt_v, sem_in)

    def fill_ones(i, carry):
        ones_v[pl.ds(i * LANES, LANES)] = jnp.ones((LANES,), jnp.float32)
        return carry

    lax.fori_loop(0, CHUNK // LANES, fill_ones, 0)

    def fill_zero(i, carry):
        zbuf[pl.ds(i * LANES, LANES)] = jnp.zeros((LANES,), jnp.float32)
        return carry

    lax.fori_loop(0, RPT // LANES, fill_zero, 0)
    pltpu.sync_copy(zbuf, acc_sh.at[pl.ds(s * RPT, RPT)])
    in_cp.wait()
    t_cp.wait()
    plsc.subcore_barrier()

    def fire(j, carry):
        pltpu.async_copy(ones_v, acc_sh.at[dst_v.at[j]], sem_sc, add=True)
        return carry

    lax.fori_loop(0, NCH, fire, 0)

    @pl.when(wid < XTRA)
    def _tail():
        pltpu.async_copy(ones_v, acc_sh.at[dstt_v.at[0]], sem_sc, add=True)

    def drain(j, carry):
        pltpu.make_async_copy(ones_v, acc_sh.at[dst_v.at[0]], sem_sc).wait()
        return carry

    lax.fori_loop(0, NCH, drain, 0)

    @pl.when(wid < XTRA)
    def _tail_drain():
        pltpu.make_async_copy(ones_v, acc_sh.at[dst_v.at[0]], sem_sc).wait()

    plsc.subcore_barrier()
    pltpu.sync_copy(acc_sh.at[pl.ds(s * RPT, RPT)],
                    degp.at[c, pl.ds(s * RPT, RPT)])


def _sc_degree(eidx):
    kern = pl.kernel(
        _deg_body,
        out_type=jax.ShapeDtypeStruct((NC, N_PAD), jnp.float32),
        mesh=_mesh(),
        scratch_types=[
            pltpu.VMEM_SHARED((N_PAD,), jnp.float32),
            pltpu.VMEM((NCH, CHUNK), jnp.int32),
            pltpu.VMEM((1, CHUNK), jnp.int32),
            pltpu.VMEM((CHUNK,), jnp.float32),
            pltpu.VMEM((RPT,), jnp.float32),
            pltpu.SemaphoreType.DMA,
            pltpu.SemaphoreType.DMA,
        ],
        compiler_params=_SC_PARAMS,
    )
    return kern(eidx)


def _layer_body(eidx, table, outp, table_sh, acc_sh, src_v, dst_v, srct_v,
                dstt_v, rows_v, rowt_v, sem_i, sem_g, sem_s):
    c = lax.axis_index("c")
    s = lax.axis_index("s")
    wid = c * NS + s

    si_cp = pltpu.async_copy(eidx.at[0, pl.ds(wid * NCH, NCH)], src_v, sem_i)
    di_cp = pltpu.async_copy(eidx.at[1, pl.ds(wid * NCH, NCH)], dst_v, sem_i)
    trow = NW * NCH + jnp.minimum(wid, XTRA - 1)
    st_cp = pltpu.async_copy(eidx.at[0, pl.ds(trow, 1)], srct_v, sem_i)
    dt_cp = pltpu.async_copy(eidx.at[1, pl.ds(trow, 1)], dstt_v, sem_i)
    pltpu.sync_copy(table.at[pl.ds(s * RPT, RPT)],
                    table_sh.at[pl.ds(s * RPT, RPT)])
    pltpu.sync_copy(table.at[pl.ds(s * RPT, RPT)],
                    acc_sh.at[pl.ds(s * RPT, RPT)])
    si_cp.wait()
    di_cp.wait()
    st_cp.wait()
    dt_cp.wait()
    plsc.subcore_barrier()

    def fire_g(g, half):
        for k in range(KG):
            pltpu.async_copy(table_sh.at[src_v.at[g * KG + k]],
                             rows_v.at[half * KG + k], sem_g)

    def drain_g():
        for _ in range(KG):
            pltpu.make_async_copy(table_sh.at[src_v.at[0]], rows_v.at[0],
                                  sem_g).wait()

    def fire_s(g, half):
        for k in range(KG):
            pltpu.async_copy(rows_v.at[half * KG + k],
                             acc_sh.at[dst_v.at[g * KG + k]], sem_s,
                             add=True)

    def drain_s():
        for _ in range(KG):
            pltpu.make_async_copy(rows_v.at[0], acc_sh.at[dst_v.at[0]],
                                  sem_s).wait()

    fire_g(0, 0)

    def body(t, carry):
        g = 2 * t
        drain_g()
        fire_g(g + 1, 1)
        fire_s(g, 0)
        drain_s()
        drain_g()

        @pl.when(g + 2 < NGRP)
        def _():
            fire_g(g + 2, 0)

        fire_s(g + 1, 1)
        drain_s()
        return carry

    lax.fori_loop(0, NGRP // 2, body, 0)

    @pl.when(wid < XTRA)
    def _tail():
        pltpu.async_copy(table_sh.at[srct_v.at[0]], rowt_v, sem_g).wait()
        pltpu.async_copy(rowt_v, acc_sh.at[dstt_v.at[0]], sem_s,
                         add=True).wait()

    plsc.subcore_barrier()
    pltpu.sync_copy(acc_sh.at[pl.ds(s * RPT, RPT)],
                    outp.at[c, pl.ds(s * RPT, RPT)])


def _sc_layer1(eidx, table):
    kern = pl.kernel(
        _layer_body,
        out_type=jax.ShapeDtypeStruct((NC, N_PAD, HID_DIM), jnp.float32),
        mesh=_mesh(),
        scratch_types=[
            pltpu.VMEM_SHARED((N_PAD, HID_DIM), jnp.float32),
            pltpu.VMEM_SHARED((N_PAD, HID_DIM), jnp.float32),
            pltpu.VMEM((NCH, CHUNK), jnp.int32),
            pltpu.VMEM((NCH, CHUNK), jnp.int32),
            pltpu.VMEM((1, CHUNK), jnp.int32),
            pltpu.VMEM((1, CHUNK), jnp.int32),
            pltpu.VMEM((2 * KG, CHUNK, HID_DIM), jnp.float32),
            pltpu.VMEM((CHUNK, HID_DIM), jnp.float32),
            pltpu.SemaphoreType.DMA,
            pltpu.SemaphoreType.DMA,
            pltpu.SemaphoreType.DMA,
        ],
        compiler_params=_SC_PARAMS,
    )
    return kern(eidx, table)


def _flat_body(eidx, table, outp, table_sh, acc_sh, src_v, dst_v, srct_v,
               dstt_v, rows_v, rowt_v, sem_i, sem_g, sem_s):
    c = lax.axis_index("c")
    s = lax.axis_index("s")
    wid = c * NS + s
    rpt = (N_PAD * OUT_DIM) // NS

    si_cp = pltpu.async_copy(eidx.at[0, pl.ds(wid * NCH, NCH)], src_v, sem_i)
    di_cp = pltpu.async_copy(eidx.at[1, pl.ds(wid * NCH, NCH)], dst_v, sem_i)
    trow = NW * NCH + jnp.minimum(wid, XTRA - 1)
    st_cp = pltpu.async_copy(eidx.at[0, pl.ds(trow, 1)], srct_v, sem_i)
    dt_cp = pltpu.async_copy(eidx.at[1, pl.ds(trow, 1)], dstt_v, sem_i)
    pltpu.sync_copy(table.at[pl.ds(s * rpt, rpt)],
                    table_sh.at[pl.ds(s * rpt, rpt)])
    pltpu.sync_copy(table.at[pl.ds(s * rpt, rpt)],
                    acc_sh.at[pl.ds(s * rpt, rpt)])
    si_cp.wait()
    di_cp.wait()
    st_cp.wait()
    dt_cp.wait()
    plsc.subcore_barrier()

    def fire_g(g, half):
        for k in range(KG):
            j = g * KG + k
            for h in range(OUT_DIM):
                pltpu.async_copy(
                    table_sh.at[pl.ds(h * N_PAD, N_PAD)].at[src_v.at[j]],
                    rows_v.at[half * KG + k, h], sem_g)

    def drain_g():
        for _ in range(OUT_DIM * KG):
            pltpu.make_async_copy(
                table_sh.at[pl.ds(0, N_PAD)].at[src_v.at[0]],
                rows_v.at[0, 0], sem_g).wait()

    def fire_s(g, half):
        for k in range(KG):
            j = g * KG + k
            for h in range(OUT_DIM):
                pltpu.async_copy(
                    rows_v.at[half * KG + k, h],
                    acc_sh.at[pl.ds(h * N_PAD, N_PAD)].at[dst_v.at[j]],
                    sem_s, add=True)

    def drain_s():
        for _ in range(OUT_DIM * KG):
            pltpu.make_async_copy(
                rows_v.at[0, 0],
                acc_sh.at[pl.ds(0, N_PAD)].at[dst_v.at[0]], sem_s).wait()

    fire_g(0, 0)

    def body(t, carry):
        g = 2 * t
        drain_g()
        fire_g(g + 1, 1)
        fire_s(g, 0)
        drain_s()
        drain_g()

        @pl.when(g + 2 < NGRP)
        def _():
            fire_g(g + 2, 0)

        fire_s(g + 1, 1)
        drain_s()
        return carry

    lax.fori_loop(0, NGRP // 2, body, 0)

    @pl.when(wid < XTRA)
    def _tail():
        for h in range(OUT_DIM):
            pltpu.async_copy(
                table_sh.at[pl.ds(h * N_PAD, N_PAD)].at[srct_v.at[0]],
                rowt_v.at[h], sem_g).wait()
            pltpu.async_copy(
                rowt_v.at[h],
                acc_sh.at[pl.ds(h * N_PAD, N_PAD)].at[dstt_v.at[0]], sem_s,
                add=True).wait()

    plsc.subcore_barrier()
    pltpu.sync_copy(acc_sh.at[pl.ds(s * rpt, rpt)],
                    outp.at[c, pl.ds(s * rpt, rpt)])


def _sc_layer2(eidx, table_flat):
    m = N_PAD * OUT_DIM
    kern = pl.kernel(
        _flat_body,
        out_type=jax.ShapeDtypeStruct((NC, m), jnp.float32),
        mesh=_mesh(),
        scratch_types=[
            pltpu.VMEM_SHARED((m,), jnp.float32),
            pltpu.VMEM_SHARED((m,), jnp.float32),
            pltpu.VMEM((NCH, CHUNK), jnp.int32),
            pltpu.VMEM((NCH, CHUNK), jnp.int32),
            pltpu.VMEM((1, CHUNK), jnp.int32),
            pltpu.VMEM((1, CHUNK), jnp.int32),
            pltpu.VMEM((2 * KG, OUT_DIM, CHUNK), jnp.float32),
            pltpu.VMEM((OUT_DIM, CHUNK), jnp.float32),
            pltpu.SemaphoreType.DMA,
            pltpu.SemaphoreType.DMA,
            pltpu.SemaphoreType.DMA,
        ],
        compiler_params=_SC_PARAMS,
    )
    return kern(eidx, table_flat)


def _tc1a_body(x_ref, w1_ref, h_ref):
    h_ref[pl.ds(0, N_NODES), :] = jnp.dot(
        x_ref[...], w1_ref[...], preferred_element_type=jnp.float32)


def _tc1a(x, W1):
    return pl.pallas_call(
        _tc1a_body,
        out_shape=jax.ShapeDtypeStruct((N_PAD, HID_DIM), jnp.float32),
    )(x, W1)


def _tc1b_body(h_ref, degp_ref, g1_ref, dinv_ref, dinvt_ref):
    deg = degp_ref[0] + degp_ref[1] + 1.0
    dt = lax.rsqrt(deg)
    dinvt_ref[...] = dt[None, :]
    dsub = dt[:, None]
    dinv_ref[...] = dsub
    g1_ref[...] = h_ref[...] * dsub


def _tc1b(h, degp):
    return pl.pallas_call(
        _tc1b_body,
        out_shape=[
            jax.ShapeDtypeStruct((N_PAD, HID_DIM), jnp.float32),
            jax.ShapeDtypeStruct((N_PAD, 1), jnp.float32),
            jax.ShapeDtypeStruct((1, N_PAD), jnp.float32),
        ],
    )(h, degp)


def _tc2_body(s1p_ref, g1_ref, dinv_ref, dinvt_ref, b1_ref, w2_ref, g2_ref):
    ssum = s1p_ref[0] + s1p_ref[1] - g1_ref[...]
    h1o = jnp.maximum(ssum * dinv_ref[...] + b1_ref[...][None, :], 0.0)
    g2t = lax.dot_general(
        w2_ref[...].T, h1o, (((1,), (1,)), ((), ())),
        preferred_element_type=jnp.float32) * dinvt_ref[...]
    g2_ref[pl.ds(0, N_PAD)] = g2t[0]
    g2_ref[pl.ds(N_PAD, N_PAD)] = g2t[1]


def _tc2(s1p, g1, dinv, dinvt, b1, W2):
    return pl.pallas_call(
        _tc2_body,
        out_shape=jax.ShapeDtypeStruct((N_PAD * OUT_DIM,), jnp.float32),
    )(s1p, g1, dinv, dinvt, b1, W2)


def _tc3_body(s2p_ref, g2_ref, dinvt_ref, b2_ref, out_ref):
    se = (s2p_ref[0, pl.ds(0, N_PAD)] + s2p_ref[1, pl.ds(0, N_PAD)]
          - g2_ref[pl.ds(0, N_PAD)])
    so = (s2p_ref[0, pl.ds(N_PAD, N_PAD)] + s2p_ref[1, pl.ds(N_PAD, N_PAD)]
          - g2_ref[pl.ds(N_PAD, N_PAD)])
    outt = (jnp.stack([se, so], axis=0) * dinvt_ref[...]
            + b2_ref[...][:, None])
    out_ref[...] = outt[:, :N_NODES].T


def _tc3(s2p, g2_flat, dinvt, b2):
    return pl.pallas_call(
        _tc3_body,
        out_shape=jax.ShapeDtypeStruct((N_NODES, OUT_DIM), jnp.float32),
    )(s2p, g2_flat, dinvt, b2)


def kernel(x, edge_index, W1, b1, W2, b2):
    eidx = edge_index.astype(jnp.int32).reshape(2, ROWS, CHUNK)

    degp = _sc_degree(eidx)
    h1 = _tc1a(x, W1)
    g1, dinv, dinvt = _tc1b(h1, degp)
    s1p = _sc_layer1(eidx, g1)
    g2 = _tc2(s1p, g1, dinv, dinvt, b1, W2)
    s2p = _sc_layer2(eidx, g2)
    return _tc3(s2p, g2, dinvt, b2)
